# baseline (speedup 1.0000x reference)
import functools

import jax
import jax.numpy as jnp
from jax import lax
from jax.experimental import pallas as pl
from jax.experimental.pallas import tpu as pltpu
from jax.experimental.pallas import tpu_sc as plsc


def _tc_in_body(x_ref, w_ref, b_ref, o_ref):
    o_ref[...] = jax.nn.relu(
        jnp.dot(x_ref[...], w_ref[...], preferred_element_type=jnp.float32)
        + b_ref[...])


def _tc_in(x, W_in, b_in):
    N, F = x.shape
    H = W_in.shape[1]
    blk = 2000
    return pl.pallas_call(
        _tc_in_body,
        grid=(N // blk,),
        in_specs=[
            pl.BlockSpec((blk, F), lambda i: (i, 0)),
            pl.BlockSpec((F, H), lambda i: (0, 0)),
            pl.BlockSpec((1, H), lambda i: (0, 0)),
        ],
        out_specs=pl.BlockSpec((blk, H), lambda i: (i, 0)),
        out_shape=jax.ShapeDtypeStruct((N, H), jnp.float32),
    )(x, W_in, b_in.reshape(1, -1))


def _tc_mid_body(o1_ref, dinv_ref, w_ref, g_ref):
    t = o1_ref[...] * dinv_ref[...]
    h = jax.nn.relu(jnp.dot(t, w_ref[...], preferred_element_type=jnp.float32))
    g_ref[...] = h * dinv_ref[...]


def _tc_mid(out1, dinv, W1):
    M, H = out1.shape
    blk = 2048
    return pl.pallas_call(
        _tc_mid_body,
        grid=(M // blk,),
        in_specs=[
            pl.BlockSpec((blk, H), lambda i: (i, 0)),
            pl.BlockSpec((blk, 1), lambda i: (i, 0)),
            pl.BlockSpec((H, H), lambda i: (0, 0)),
        ],
        out_specs=pl.BlockSpec((blk, H), lambda i: (i, 0)),
        out_shape=jax.ShapeDtypeStruct((M, H), jnp.float32),
    )(out1, dinv.reshape(M, 1), W1)


def _tc_out_body(o2_ref, dinv_ref, w2_ref, wo_ref, bo_ref, y_ref, *, maxN):
    i = pl.program_id(0)
    t = o2_ref[...] * dinv_ref[...]
    h2 = jax.nn.relu(jnp.dot(t, w2_ref[...], preferred_element_type=jnp.float32))
    pooled = jnp.sum(h2, axis=0, keepdims=True) * (1.0 / maxN)
    y_ref[pl.ds(i, 1), :] = (
        jnp.dot(pooled, wo_ref[...], preferred_element_type=jnp.float32)
        + bo_ref[...])


def _tc_out(out2, dinv, W2, W_out, b_out, B, maxN):
    M, H = out2.shape
    OUT = W_out.shape[1]
    return pl.pallas_call(
        functools.partial(_tc_out_body, maxN=maxN),
        grid=(B,),
        in_specs=[
            pl.BlockSpec((maxN, H), lambda i: (i, 0)),
            pl.BlockSpec((maxN, 1), lambda i: (i, 0)),
            pl.BlockSpec((H, H), lambda i: (0, 0)),
            pl.BlockSpec((H, OUT), lambda i: (0, 0)),
            pl.BlockSpec((1, OUT), lambda i: (0, 0)),
        ],
        out_specs=pl.BlockSpec((B, OUT), lambda i: (0, 0)),
        out_shape=jax.ShapeDtypeStruct((B, OUT), jnp.float32),
    )(out2, dinv.reshape(M, 1), W2, W_out, b_out.reshape(1, -1))


def _pre_jnp(h0, subG_node, eu, ev):
    B, maxN = subG_node.shape
    M = B * maxN
    lu = jax.vmap(jnp.searchsorted)(subG_node, eu).astype(jnp.int32)
    lv = jax.vmap(jnp.searchsorted)(subG_node, ev).astype(jnp.int32)
    off = (jnp.arange(B, dtype=jnp.int32) * maxN)[:, None]
    gsrc = jnp.concatenate([lu + off, lv + off], axis=1)
    gdst = jnp.concatenate([lv + off, lu + off], axis=1)
    deg = jnp.zeros((M,), jnp.float32).at[gsrc.reshape(-1)].add(1.0)
    deg = jnp.where(deg < 0.5, deg + 1.0, deg)
    dinv = deg ** -0.5
    g0 = h0[subG_node.reshape(-1)] * dinv[:, None]
    return g0, gsrc, gdst, dinv


def _spmm_jnp(table, gsrc, gdst):
    M, H = table.shape
    return (jnp.zeros((M, H), jnp.float32)
            .at[gsrc.reshape(-1)].add(table[gdst.reshape(-1)]))


def kernel(x, subG_node, subG_edge, W_in, b_in, W1, W2, W_out, b_out):
    B, maxN = subG_node.shape
    eu = subG_edge[..., 0]
    ev = subG_edge[..., 1]
    h0 = _tc_in(x, W_in, b_in)
    g0, gsrc, gdst, dinv = _pre_jnp(h0, subG_node, eu, ev)
    out1 = _spmm_jnp(g0, gsrc, gdst)
    g1 = _tc_mid(out1, dinv, W1)
    out2 = _spmm_jnp(g1, gsrc, gdst)
    return _tc_out(out2, dinv, W2, W_out, b_out, B, maxN)


# speedup vs baseline: 63.1732x; 63.1732x over previous
import functools

import jax
import jax.numpy as jnp
from jax import lax
from jax.experimental import pallas as pl
from jax.experimental.pallas import tpu as pltpu
from jax.experimental.pallas import tpu_sc as plsc


def _tc_in_body(x_ref, w_ref, b_ref, o_ref):
    o_ref[...] = jax.nn.relu(
        jnp.dot(x_ref[...], w_ref[...], preferred_element_type=jnp.float32)
        + b_ref[...])


def _tc_in(x, W_in, b_in):
    N, F = x.shape
    H = W_in.shape[1]
    blk = 2000
    return pl.pallas_call(
        _tc_in_body,
        grid=(N // blk,),
        in_specs=[
            pl.BlockSpec((blk, F), lambda i: (i, 0)),
            pl.BlockSpec((F, H), lambda i: (0, 0)),
            pl.BlockSpec((1, H), lambda i: (0, 0)),
        ],
        out_specs=pl.BlockSpec((blk, H), lambda i: (i, 0)),
        out_shape=jax.ShapeDtypeStruct((N, H), jnp.float32),
    )(x, W_in, b_in.reshape(1, -1))


def _tc_mid_body(o1_ref, dinv_ref, w_ref, g_ref):
    t = o1_ref[...] * dinv_ref[...]
    h = jax.nn.relu(jnp.dot(t, w_ref[...], preferred_element_type=jnp.float32))
    g_ref[...] = h * dinv_ref[...]


def _tc_mid(out1, dinv, W1):
    M, H = out1.shape
    blk = 2048
    return pl.pallas_call(
        _tc_mid_body,
        grid=(M // blk,),
        in_specs=[
            pl.BlockSpec((blk, H), lambda i: (i, 0)),
            pl.BlockSpec((blk, 1), lambda i: (i, 0)),
            pl.BlockSpec((H, H), lambda i: (0, 0)),
        ],
        out_specs=pl.BlockSpec((blk, H), lambda i: (i, 0)),
        out_shape=jax.ShapeDtypeStruct((M, H), jnp.float32),
    )(out1, dinv.reshape(M, 1), W1)


def _tc_out_body(o2_ref, dinv_ref, w2_ref, wo_ref, bo_ref, y_ref, *, maxN):
    i = pl.program_id(0)
    t = o2_ref[...] * dinv_ref[...]
    h2 = jax.nn.relu(jnp.dot(t, w2_ref[...], preferred_element_type=jnp.float32))
    pooled = jnp.sum(h2, axis=0, keepdims=True) * (1.0 / maxN)
    y_ref[pl.ds(i, 1), :] = (
        jnp.dot(pooled, wo_ref[...], preferred_element_type=jnp.float32)
        + bo_ref[...])


def _tc_out(out2, dinv, W2, W_out, b_out, B, maxN):
    M, H = out2.shape
    OUT = W_out.shape[1]
    return pl.pallas_call(
        functools.partial(_tc_out_body, maxN=maxN),
        grid=(B,),
        in_specs=[
            pl.BlockSpec((maxN, H), lambda i: (i, 0)),
            pl.BlockSpec((maxN, 1), lambda i: (i, 0)),
            pl.BlockSpec((H, H), lambda i: (0, 0)),
            pl.BlockSpec((H, OUT), lambda i: (0, 0)),
            pl.BlockSpec((1, OUT), lambda i: (0, 0)),
        ],
        out_specs=pl.BlockSpec((B, OUT), lambda i: (0, 0)),
        out_shape=jax.ShapeDtypeStruct((B, OUT), jnp.float32),
    )(out2, dinv.reshape(M, 1), W2, W_out, b_out.reshape(1, -1))


_L = 16


def _rsqrt16(d):
    i = plsc.bitcast(d, jnp.int32)
    i = jnp.int32(0x5F3759DF) - lax.shift_right_logical(i, 1)
    y = plsc.bitcast(i, jnp.float32)
    for _ in range(3):
        y = y * (1.5 - 0.5 * d * y * y)
    return y


def _sc_pre_body(h0, nodes, nodes3, eu, ev,
                 g0, gsrc, gdst, dinv,
                 nodes_v, nodes2_v, pos_v, eu_v, ev_v, deg_v, dinv_v,
                 gsrc_v, gdst_v, rows_v, sem,
                 *, N, B, maxN, maxE, H):
    wid = lax.axis_index("s") * 2 + lax.axis_index("c")

    @pl.when(wid < B)
    def _():
        b = wid
        boff = b * maxN
        pltpu.sync_copy(nodes.at[b], nodes_v)
        pltpu.sync_copy(nodes3.at[b], nodes2_v)
        pltpu.sync_copy(eu.at[b], eu_v)
        pltpu.sync_copy(ev.at[b], ev_v)

        def pos_body(k, _):
            idx = nodes_v[pl.ds(k * _L, _L)]
            plsc.store_scatter(pos_v, [idx],
                               lax.iota(jnp.int32, _L) + k * _L)
            deg_v[pl.ds(k * _L, _L)] = jnp.zeros((_L,), jnp.float32)
            return _
        lax.fori_loop(0, maxN // _L, pos_body, None)

        ones = jnp.ones((_L,), jnp.float32)

        def edge_body(k, _):
            u = eu_v[pl.ds(k * _L, _L)]
            v = ev_v[pl.ds(k * _L, _L)]
            lu = plsc.load_gather(pos_v, [u])
            lv = plsc.load_gather(pos_v, [v])
            plsc.addupdate_scatter(deg_v, [lu], ones)
            plsc.addupdate_scatter(deg_v, [lv], ones)
            gsrc_v[pl.ds(k * _L, _L)] = lu + boff
            gsrc_v[pl.ds(maxE + k * _L, _L)] = lv + boff
            gdst_v[pl.ds(k * _L, _L)] = lv + boff
            gdst_v[pl.ds(maxE + k * _L, _L)] = lu + boff
            return _
        lax.fori_loop(0, maxE // _L, edge_body, None)

        def dinv_body(k, _):
            d = deg_v[pl.ds(k * _L, _L)]
            d = jnp.where(d < 0.5, d + 1.0, d)
            dinv_v[pl.ds(k * _L, _L)] = _rsqrt16(d)
            return _
        lax.fori_loop(0, maxN // _L, dinv_body, None)

        pltpu.sync_copy(gsrc_v, gsrc.at[b])
        pltpu.sync_copy(gdst_v, gdst.at[b])
        pltpu.sync_copy(dinv_v, dinv.at[pl.ds(b * maxN, maxN)])

        def chunk_body(c, _):
            pltpu.async_copy(h0.at[nodes2_v.at[c]], rows_v, sem).wait()

            def row_body(r, _):
                s = plsc.load_gather(
                    dinv_v, [jnp.full((_L,), c * 128 + r, jnp.int32)])
                for f in range(H // _L):
                    rows_v[r, pl.ds(f * _L, _L)] = (
                        rows_v[r, pl.ds(f * _L, _L)] * s)
                return _
            lax.fori_loop(0, 128, row_body, None)
            pltpu.sync_copy(rows_v, g0.at[pl.ds(boff + c * 128, 128)])
            return _
        lax.fori_loop(0, maxN // 128, chunk_body, None)


def _sc_pre(h0, subG_node, eu, ev):
    N, H = h0.shape
    B, maxN = subG_node.shape
    maxE = eu.shape[1]
    M = B * maxN
    mesh = plsc.VectorSubcoreMesh(core_axis_name="c", subcore_axis_name="s")
    f = pl.kernel(
        functools.partial(_sc_pre_body, N=N, B=B, maxN=maxN, maxE=maxE, H=H),
        mesh=mesh,
        compiler_params=pltpu.CompilerParams(needs_layout_passes=False),
        out_type=(
            jax.ShapeDtypeStruct((M, H), jnp.float32),
            jax.ShapeDtypeStruct((B, 2 * maxE), jnp.int32),
            jax.ShapeDtypeStruct((B, 2 * maxE), jnp.int32),
            jax.ShapeDtypeStruct((M,), jnp.float32),
        ),
        scratch_types=[
            pltpu.VMEM((maxN,), jnp.int32),
            pltpu.VMEM((maxN // 128, 128), jnp.int32),
            pltpu.VMEM((N,), jnp.int32),
            pltpu.VMEM((maxE,), jnp.int32),
            pltpu.VMEM((maxE,), jnp.int32),
            pltpu.VMEM((maxN,), jnp.float32),
            pltpu.VMEM((maxN,), jnp.float32),
            pltpu.VMEM((2 * maxE,), jnp.int32),
            pltpu.VMEM((2 * maxE,), jnp.int32),
            pltpu.VMEM((128, H), jnp.float32),
            pltpu.SemaphoreType.DMA,
        ],
    )
    return f(h0, subG_node, subG_node.reshape(B, maxN // 128, 128), eu, ev)


def _sc_spmm_body(table, gsrc, gdst, out,
                  idxd_v, idxs_v, rows_v, zbuf_v, acc_sh, sem,
                  *, B, maxN, maxE, H, grp):
    cid = lax.axis_index("c")
    sid = lax.axis_index("s")
    npass = B // (2 * grp)
    inc = 2 * maxE
    per_tile = inc // 16
    zrows = zbuf_v.shape[0]

    def zb(r, _):
        for f in range(H // _L):
            zbuf_v[r, pl.ds(f * _L, _L)] = jnp.zeros((_L,), jnp.float32)
        return _
    lax.fori_loop(0, zrows, zb, None)

    for p in range(npass):
        base_sub = p * 2 * grp + cid * grp
        base_row = base_sub * maxN

        myrows = grp * maxN // 16
        def zacc(j, _):
            pltpu.sync_copy(
                zbuf_v, acc_sh.at[pl.ds(sid * myrows + j * zrows, zrows)])
            return _
        lax.fori_loop(0, myrows // zrows, zacc, None)
        plsc.subcore_barrier()

        def sub_body(q, _):
            b = base_sub + q

            def chunk_body(j, _):
                off = sid * per_tile + j * 128
                pltpu.sync_copy(gdst.at[b, pl.ds(off, 128)], idxd_v)
                pltpu.sync_copy(gsrc.at[b, pl.ds(off, 128)], idxs_v)
                cp = pltpu.async_copy(table.at[idxd_v], rows_v, sem)
                for t in range(128 // _L):
                    idxs_v[pl.ds(t * _L, _L)] = (
                        idxs_v[pl.ds(t * _L, _L)] - base_row)
                cp.wait()
                pltpu.sync_copy(rows_v, acc_sh.at[idxs_v], add=True)
                return _
            lax.fori_loop(0, per_tile // 128, chunk_body, None)
            return _
        lax.fori_loop(0, grp, sub_body, None)
        plsc.subcore_barrier()

        def wb(j, _):
            r0 = sid * myrows + j * 128
            pltpu.sync_copy(acc_sh.at[pl.ds(r0, 128)],
                            out.at[pl.ds(base_row + r0, 128)])
            return _
        lax.fori_loop(0, myrows // 128, wb, None)
        plsc.subcore_barrier()


def _sc_spmm(table, gsrc, gdst, B, maxN, maxE):
    M, H = table.shape
    grp = 4
    mesh = plsc.VectorSubcoreMesh(core_axis_name="c", subcore_axis_name="s")
    f = pl.kernel(
        functools.partial(_sc_spmm_body, B=B, maxN=maxN, maxE=maxE, H=H,
                          grp=grp),
        mesh=mesh,
        compiler_params=pltpu.CompilerParams(needs_layout_passes=False),
        out_type=jax.ShapeDtypeStruct((M, H), jnp.float32),
        scratch_types=[
            pltpu.VMEM((128,), jnp.int32),
            pltpu.VMEM((128,), jnp.int32),
            pltpu.VMEM((128, H), jnp.float32),
            pltpu.VMEM((64, H), jnp.float32),
            pltpu.VMEM_SHARED((grp * maxN, H), jnp.float32),
            pltpu.SemaphoreType.DMA,
        ],
    )
    return f(table, gsrc, gdst)


def kernel(x, subG_node, subG_edge, W_in, b_in, W1, W2, W_out, b_out):
    B, maxN = subG_node.shape
    maxE = subG_edge.shape[1]
    eu = subG_edge[..., 0]
    ev = subG_edge[..., 1]
    h0 = _tc_in(x, W_in, b_in)
    g0, gsrc, gdst, dinv = _sc_pre(h0, subG_node, eu, ev)
    out1 = _sc_spmm(g0, gsrc, gdst, B, maxN, maxE)
    g1 = _tc_mid(out1, dinv, W1)
    out2 = _sc_spmm(g1, gsrc, gdst, B, maxN, maxE)
    return _tc_out(out2, dinv, W2, W_out, b_out, B, maxN)


# speedup vs baseline: 94.7273x; 1.4995x over previous
import functools

import jax
import jax.numpy as jnp
from jax import lax
from jax.experimental import pallas as pl
from jax.experimental.pallas import tpu as pltpu
from jax.experimental.pallas import tpu_sc as plsc


def _tc_in_body(x_ref, w_ref, b_ref, o_ref):
    o_ref[...] = jax.nn.relu(
        jnp.dot(x_ref[...], w_ref[...], preferred_element_type=jnp.float32)
        + b_ref[...])


def _tc_in(x, W_in, b_in):
    N, F = x.shape
    H = W_in.shape[1]
    blk = 2000
    return pl.pallas_call(
        _tc_in_body,
        grid=(N // blk,),
        in_specs=[
            pl.BlockSpec((blk, F), lambda i: (i, 0)),
            pl.BlockSpec((F, H), lambda i: (0, 0)),
            pl.BlockSpec((1, H), lambda i: (0, 0)),
        ],
        out_specs=pl.BlockSpec((blk, H), lambda i: (i, 0)),
        out_shape=jax.ShapeDtypeStruct((N, H), jnp.float32),
    )(x, W_in, b_in.reshape(1, -1))


def _tc_mid_body(o1_ref, dinv_ref, w_ref, g_ref):
    t = o1_ref[...] * dinv_ref[...]
    h = jax.nn.relu(jnp.dot(t, w_ref[...], preferred_element_type=jnp.float32))
    g_ref[...] = h * dinv_ref[...]


def _tc_mid(out1, dinv, W1):
    M, H = out1.shape
    blk = 2048
    return pl.pallas_call(
        _tc_mid_body,
        grid=(M // blk,),
        in_specs=[
            pl.BlockSpec((blk, H), lambda i: (i, 0)),
            pl.BlockSpec((blk, 1), lambda i: (i, 0)),
            pl.BlockSpec((H, H), lambda i: (0, 0)),
        ],
        out_specs=pl.BlockSpec((blk, H), lambda i: (i, 0)),
        out_shape=jax.ShapeDtypeStruct((M, H), jnp.float32),
    )(out1, dinv.reshape(M, 1), W1)


def _tc_out_body(o2_ref, dinv_ref, w2_ref, wo_ref, bo_ref, y_ref, *, maxN):
    i = pl.program_id(0)
    t = o2_ref[...] * dinv_ref[...]
    h2 = jax.nn.relu(jnp.dot(t, w2_ref[...], preferred_element_type=jnp.float32))
    pooled = jnp.sum(h2, axis=0, keepdims=True) * (1.0 / maxN)
    y_ref[pl.ds(i, 1), :] = (
        jnp.dot(pooled, wo_ref[...], preferred_element_type=jnp.float32)
        + bo_ref[...])


def _tc_out(out2, dinv, W2, W_out, b_out, B, maxN):
    M, H = out2.shape
    OUT = W_out.shape[1]
    return pl.pallas_call(
        functools.partial(_tc_out_body, maxN=maxN),
        grid=(B,),
        in_specs=[
            pl.BlockSpec((maxN, H), lambda i: (i, 0)),
            pl.BlockSpec((maxN, 1), lambda i: (i, 0)),
            pl.BlockSpec((H, H), lambda i: (0, 0)),
            pl.BlockSpec((H, OUT), lambda i: (0, 0)),
            pl.BlockSpec((1, OUT), lambda i: (0, 0)),
        ],
        out_specs=pl.BlockSpec((B, OUT), lambda i: (0, 0)),
        out_shape=jax.ShapeDtypeStruct((B, OUT), jnp.float32),
    )(out2, dinv.reshape(M, 1), W2, W_out, b_out.reshape(1, -1))


_L = 16
_GRP = 2


def _rsqrt16(d):
    i = plsc.bitcast(d, jnp.int32)
    i = jnp.int32(0x5F3759DF) - lax.shift_right_logical(i, 1)
    y = plsc.bitcast(i, jnp.float32)
    for _ in range(3):
        y = y * (1.5 - 0.5 * d * y * y)
    return y


def _sc_pre_body(h0, nodes, nodes3, eu, ev,
                 g0, gsrc, gdst, dinv,
                 nodes_v, nodes2_v, pos_v, eu_v, ev_v, deg_v, dinv_v,
                 gsrc_v, gdst_v, rows_v, sem,
                 *, N, B, maxN, maxE, H):
    wid = lax.axis_index("s") * 2 + lax.axis_index("c")

    @pl.when(wid < B)
    def _():
        b = wid
        boff = b * maxN
        pltpu.sync_copy(nodes.at[b], nodes_v)
        pltpu.sync_copy(nodes3.at[b], nodes2_v)
        pltpu.sync_copy(eu.at[b], eu_v)
        pltpu.sync_copy(ev.at[b], ev_v)

        def pos_body(k, _):
            idx = nodes_v[pl.ds(k * _L, _L)]
            plsc.store_scatter(pos_v, [idx],
                               lax.iota(jnp.int32, _L) + k * _L)
            deg_v[pl.ds(k * _L, _L)] = jnp.zeros((_L,), jnp.float32)
            return _
        lax.fori_loop(0, maxN // _L, pos_body, None)

        ones = jnp.ones((_L,), jnp.float32)
        qoff = (b % _GRP) * maxN

        def edge_body(k, _):
            u = eu_v[pl.ds(k * _L, _L)]
            v = ev_v[pl.ds(k * _L, _L)]
            lu = plsc.load_gather(pos_v, [u])
            lv = plsc.load_gather(pos_v, [v])
            plsc.addupdate_scatter(deg_v, [lu], ones)
            plsc.addupdate_scatter(deg_v, [lv], ones)
            gsrc_v[pl.ds(k * _L, _L)] = lu + qoff
            gsrc_v[pl.ds(maxE + k * _L, _L)] = lv + qoff
            gdst_v[pl.ds(k * _L, _L)] = lv + boff
            gdst_v[pl.ds(maxE + k * _L, _L)] = lu + boff
            return _
        lax.fori_loop(0, maxE // _L, edge_body, None)

        def dinv_body(k, _):
            d = deg_v[pl.ds(k * _L, _L)]
            d = jnp.where(d < 0.5, d + 1.0, d)
            dinv_v[pl.ds(k * _L, _L)] = _rsqrt16(d)
            return _
        lax.fori_loop(0, maxN // _L, dinv_body, None)

        pltpu.sync_copy(gsrc_v, gsrc.at[b])
        pltpu.sync_copy(gdst_v, gdst.at[b])
        pltpu.sync_copy(dinv_v, dinv.at[pl.ds(b * maxN, maxN)])

        def chunk_body(c, _):
            pltpu.async_copy(h0.at[nodes2_v.at[c]], rows_v, sem).wait()

            def row_body(r, _):
                s = plsc.load_gather(
                    dinv_v, [jnp.full((_L,), c * 128 + r, jnp.int32)])
                for f in range(H // _L):
                    rows_v[r, pl.ds(f * _L, _L)] = (
                        rows_v[r, pl.ds(f * _L, _L)] * s)
                return _
            lax.fori_loop(0, 128, row_body, None)
            pltpu.sync_copy(rows_v, g0.at[pl.ds(boff + c * 128, 128)])
            return _
        lax.fori_loop(0, maxN // 128, chunk_body, None)


def _sc_pre(h0, subG_node, eu, ev):
    N, H = h0.shape
    B, maxN = subG_node.shape
    maxE = eu.shape[1]
    M = B * maxN
    mesh = plsc.VectorSubcoreMesh(core_axis_name="c", subcore_axis_name="s")
    f = pl.kernel(
        functools.partial(_sc_pre_body, N=N, B=B, maxN=maxN, maxE=maxE, H=H),
        mesh=mesh,
        compiler_params=pltpu.CompilerParams(needs_layout_passes=False),
        out_type=(
            jax.ShapeDtypeStruct((M, H), jnp.float32),
            jax.ShapeDtypeStruct((B, 2 * maxE), jnp.int32),
            jax.ShapeDtypeStruct((B, 2 * maxE), jnp.int32),
            jax.ShapeDtypeStruct((M,), jnp.float32),
        ),
        scratch_types=[
            pltpu.VMEM((maxN,), jnp.int32),
            pltpu.VMEM((maxN // 128, 128), jnp.int32),
            pltpu.VMEM((N,), jnp.int32),
            pltpu.VMEM((maxE,), jnp.int32),
            pltpu.VMEM((maxE,), jnp.int32),
            pltpu.VMEM((maxN,), jnp.float32),
            pltpu.VMEM((maxN,), jnp.float32),
            pltpu.VMEM((2 * maxE,), jnp.int32),
            pltpu.VMEM((2 * maxE,), jnp.int32),
            pltpu.VMEM((128, H), jnp.float32),
            pltpu.SemaphoreType.DMA,
        ],
    )
    return f(h0, subG_node, subG_node.reshape(B, maxN // 128, 128), eu, ev)


def _sc_spmm_body(table, gdst4, gsrc4, out,
                  idxd_v, idxs_v, r0_v, r1_v, r2_v, r3_v, zbuf_v, acc_sh,
                  g0_s, g1_s, g2_s, g3_s, s0_s, s1_s, s2_s, s3_s,
                  *, B, maxN, maxE, H, grp):
    cid = lax.axis_index("c")
    sid = lax.axis_index("s")
    npass = B // (2 * grp)
    nch = grp * (2 * maxE) // 16 // 128
    rows = (r0_v, r1_v, r2_v, r3_v)
    gsem = (g0_s, g1_s, g2_s, g3_s)
    ssem = (s0_s, s1_s, s2_s, s3_s)
    zrows = zbuf_v.shape[0]
    myrows = grp * maxN // 16

    def fire_gather(c, slot):
        pltpu.async_copy(table.at[idxd_v.at[c]], rows[slot], gsem[slot])

    def wait_gather(c, slot):
        pltpu.make_async_copy(
            table.at[idxd_v.at[c]], rows[slot], gsem[slot]).wait()

    def fire_scatter(c, slot):
        pltpu.async_copy(rows[slot], acc_sh.at[idxs_v.at[c]], ssem[slot],
                         add=True)

    def wait_scatter(c, slot):
        pltpu.make_async_copy(
            rows[slot], acc_sh.at[idxs_v.at[c]], ssem[slot]).wait()

    def zb(r, _):
        for f in range(H // _L):
            zbuf_v[r, pl.ds(f * _L, _L)] = jnp.zeros((_L,), jnp.float32)
        return _
    lax.fori_loop(0, zrows, zb, None)

    for p in range(npass):
        base_sub = p * 2 * grp + cid * grp
        base_row = base_sub * maxN

        for q in range(grp):
            pltpu.sync_copy(gdst4.at[base_sub + q, sid],
                            idxd_v.at[pl.ds(q * 8, 8)])
            pltpu.sync_copy(gsrc4.at[base_sub + q, sid],
                            idxs_v.at[pl.ds(q * 8, 8)])

        def zacc(j, _):
            pltpu.sync_copy(
                zbuf_v, acc_sh.at[pl.ds(sid * myrows + j * zrows, zrows)])
            return _
        lax.fori_loop(0, myrows // zrows, zacc, None)
        plsc.subcore_barrier()

        fire_gather(0, 0)
        fire_gather(1, 1)

        def grp_body(g, _):
            for s in range(4):
                c = g * 4 + s
                wait_gather(c, s)
                fire_scatter(c, s)
                t = (s + 2) % 4
                c2 = c + 2

                @pl.when(c2 < nch)
                def _():
                    @pl.when(c >= 2)
                    def _():
                        wait_scatter(c - 2, t)
                    fire_gather(c2, t)
            return _
        lax.fori_loop(0, nch // 4, grp_body, None)

        for s in range(4):
            wait_scatter(nch - 4 + s, s)
        plsc.subcore_barrier()

        pltpu.sync_copy(acc_sh.at[pl.ds(sid * myrows, myrows)],
                        out.at[pl.ds(base_row + sid * myrows, myrows)])
        plsc.subcore_barrier()


def _sc_spmm(table, gdst4, gsrc4, B, maxN, maxE):
    M, H = table.shape
    grp = _GRP
    nch = grp * (2 * maxE) // 16 // 128
    mesh = plsc.VectorSubcoreMesh(core_axis_name="c", subcore_axis_name="s")
    f = pl.kernel(
        functools.partial(_sc_spmm_body, B=B, maxN=maxN, maxE=maxE, H=H,
                          grp=grp),
        mesh=mesh,
        compiler_params=pltpu.CompilerParams(needs_layout_passes=False),
        out_type=jax.ShapeDtypeStruct((M, H), jnp.float32),
        scratch_types=[
            pltpu.VMEM((nch, 128), jnp.int32),
            pltpu.VMEM((nch, 128), jnp.int32),
            pltpu.VMEM((128, H), jnp.float32),
            pltpu.VMEM((128, H), jnp.float32),
            pltpu.VMEM((128, H), jnp.float32),
            pltpu.VMEM((128, H), jnp.float32),
            pltpu.VMEM((64, H), jnp.float32),
            pltpu.VMEM_SHARED((grp * maxN, H), jnp.float32),
            pltpu.SemaphoreType.DMA, pltpu.SemaphoreType.DMA,
            pltpu.SemaphoreType.DMA, pltpu.SemaphoreType.DMA,
            pltpu.SemaphoreType.DMA, pltpu.SemaphoreType.DMA,
            pltpu.SemaphoreType.DMA, pltpu.SemaphoreType.DMA,
        ],
    )
    return f(table, gdst4, gsrc4)


def kernel(x, subG_node, subG_edge, W_in, b_in, W1, W2, W_out, b_out):
    B, maxN = subG_node.shape
    maxE = subG_edge.shape[1]
    eu = subG_edge[..., 0]
    ev = subG_edge[..., 1]
    h0 = _tc_in(x, W_in, b_in)
    g0, gsrc, gdst, dinv = _sc_pre(h0, subG_node, eu, ev)
    gsrc4 = gsrc.reshape(B, 16, (2 * maxE) // 16 // 128, 128)
    gdst4 = gdst.reshape(B, 16, (2 * maxE) // 16 // 128, 128)
    out1 = _sc_spmm(g0, gdst4, gsrc4, B, maxN, maxE)
    g1 = _tc_mid(out1, dinv, W1)
    out2 = _sc_spmm(g1, gdst4, gsrc4, B, maxN, maxE)
    return _tc_out(out2, dinv, W2, W_out, b_out, B, maxN)


# speedup vs baseline: 97.3778x; 1.0280x over previous
import functools

import jax
import jax.numpy as jnp
from jax import lax
from jax.experimental import pallas as pl
from jax.experimental.pallas import tpu as pltpu
from jax.experimental.pallas import tpu_sc as plsc


def _tc_in_body(x_ref, w_ref, b_ref, o_ref):
    o_ref[...] = jax.nn.relu(
        jnp.dot(x_ref[...], w_ref[...], preferred_element_type=jnp.float32)
        + b_ref[...])


def _tc_in(x, W_in, b_in):
    N, F = x.shape
    H = W_in.shape[1]
    blk = 2000
    return pl.pallas_call(
        _tc_in_body,
        grid=(N // blk,),
        in_specs=[
            pl.BlockSpec((blk, F), lambda i: (i, 0)),
            pl.BlockSpec((F, H), lambda i: (0, 0)),
            pl.BlockSpec((1, H), lambda i: (0, 0)),
        ],
        out_specs=pl.BlockSpec((blk, H), lambda i: (i, 0)),
        out_shape=jax.ShapeDtypeStruct((N, H), jnp.float32),
    )(x, W_in, b_in.reshape(1, -1))


def _tc_mid_body(o1_ref, dinv_ref, w_ref, g_ref):
    t = o1_ref[...] * dinv_ref[...]
    h = jax.nn.relu(jnp.dot(t, w_ref[...], preferred_element_type=jnp.float32))
    g_ref[...] = h * dinv_ref[...]


def _tc_mid(out1, dinv, W1):
    M, H = out1.shape
    blk = 2048
    return pl.pallas_call(
        _tc_mid_body,
        grid=(M // blk,),
        in_specs=[
            pl.BlockSpec((blk, H), lambda i: (i, 0)),
            pl.BlockSpec((blk, 1), lambda i: (i, 0)),
            pl.BlockSpec((H, H), lambda i: (0, 0)),
        ],
        out_specs=pl.BlockSpec((blk, H), lambda i: (i, 0)),
        out_shape=jax.ShapeDtypeStruct((M, H), jnp.float32),
    )(out1, dinv.reshape(M, 1), W1)


def _tc_out_body(o2_ref, dinv_ref, w2_ref, wo_ref, bo_ref, y_ref, *, maxN):
    i = pl.program_id(0)
    t = o2_ref[...] * dinv_ref[...]
    h2 = jax.nn.relu(jnp.dot(t, w2_ref[...], preferred_element_type=jnp.float32))
    pooled = jnp.sum(h2, axis=0, keepdims=True) * (1.0 / maxN)
    y_ref[pl.ds(i, 1), :] = (
        jnp.dot(pooled, wo_ref[...], preferred_element_type=jnp.float32)
        + bo_ref[...])


def _tc_out(out2, dinv, W2, W_out, b_out, B, maxN):
    M, H = out2.shape
    OUT = W_out.shape[1]
    return pl.pallas_call(
        functools.partial(_tc_out_body, maxN=maxN),
        grid=(B,),
        in_specs=[
            pl.BlockSpec((maxN, H), lambda i: (i, 0)),
            pl.BlockSpec((maxN, 1), lambda i: (i, 0)),
            pl.BlockSpec((H, H), lambda i: (0, 0)),
            pl.BlockSpec((H, OUT), lambda i: (0, 0)),
            pl.BlockSpec((1, OUT), lambda i: (0, 0)),
        ],
        out_specs=pl.BlockSpec((B, OUT), lambda i: (0, 0)),
        out_shape=jax.ShapeDtypeStruct((B, OUT), jnp.float32),
    )(out2, dinv.reshape(M, 1), W2, W_out, b_out.reshape(1, -1))


_L = 16
_GRP = 2


def _rsqrt16(d):
    i = plsc.bitcast(d, jnp.int32)
    i = jnp.int32(0x5F3759DF) - lax.shift_right_logical(i, 1)
    y = plsc.bitcast(i, jnp.float32)
    for _ in range(3):
        y = y * (1.5 - 0.5 * d * y * y)
    return y


def _sc_pre_body(h0, nodes, nodes3, eu, ev,
                 g0, gsrc, gdst, dinv,
                 nodes_v, nodes2_v, pos_v, eu_v, ev_v, deg_v, dinv_v,
                 gsrc_v, gdst_v, rows_v, sem,
                 *, N, B, maxN, maxE, H):
    wid = lax.axis_index("s") * 2 + lax.axis_index("c")

    @pl.when(wid < B)
    def _():
        b = wid
        boff = b * maxN
        pltpu.sync_copy(nodes.at[b], nodes_v)
        pltpu.sync_copy(nodes3.at[b], nodes2_v)
        pltpu.sync_copy(eu.at[b], eu_v)
        pltpu.sync_copy(ev.at[b], ev_v)

        def pos_body(k, _):
            idx = nodes_v[pl.ds(k * _L, _L)]
            plsc.store_scatter(pos_v, [idx],
                               lax.iota(jnp.int32, _L) + k * _L)
            deg_v[pl.ds(k * _L, _L)] = jnp.zeros((_L,), jnp.float32)
            return _
        lax.fori_loop(0, maxN // _L, pos_body, None)

        ones = jnp.ones((_L,), jnp.float32)
        qoff = (b % _GRP) * maxN

        def edge_body(k, _):
            u = eu_v[pl.ds(k * _L, _L)]
            v = ev_v[pl.ds(k * _L, _L)]
            lu = plsc.load_gather(pos_v, [u])
            lv = plsc.load_gather(pos_v, [v])
            plsc.addupdate_scatter(deg_v, [lu], ones)
            plsc.addupdate_scatter(deg_v, [lv], ones)
            gsrc_v[pl.ds(k * _L, _L)] = lu + qoff
            gsrc_v[pl.ds(maxE + k * _L, _L)] = lv + qoff
            gdst_v[pl.ds(k * _L, _L)] = lv + boff
            gdst_v[pl.ds(maxE + k * _L, _L)] = lu + boff
            return _
        lax.fori_loop(0, maxE // _L, edge_body, None)

        def dinv_body(k, _):
            d = deg_v[pl.ds(k * _L, _L)]
            d = jnp.where(d < 0.5, d + 1.0, d)
            dinv_v[pl.ds(k * _L, _L)] = _rsqrt16(d)
            return _
        lax.fori_loop(0, maxN // _L, dinv_body, None)

        pltpu.sync_copy(gsrc_v, gsrc.at[b])
        pltpu.sync_copy(gdst_v, gdst.at[b])
        pltpu.sync_copy(dinv_v, dinv.at[pl.ds(b * maxN, maxN)])

        def chunk_body(c, _):
            pltpu.async_copy(h0.at[nodes2_v.at[c]], rows_v, sem).wait()

            def row_body(r, _):
                s = plsc.load_gather(
                    dinv_v, [jnp.full((_L,), c * 128 + r, jnp.int32)])
                for f in range(H // _L):
                    rows_v[r, pl.ds(f * _L, _L)] = (
                        rows_v[r, pl.ds(f * _L, _L)] * s)
                return _
            lax.fori_loop(0, 128, row_body, None)
            pltpu.sync_copy(rows_v, g0.at[pl.ds(boff + c * 128, 128)])
            return _
        lax.fori_loop(0, maxN // 128, chunk_body, None)


def _sc_pre(h0, subG_node, eu, ev):
    N, H = h0.shape
    B, maxN = subG_node.shape
    maxE = eu.shape[1]
    M = B * maxN
    mesh = plsc.VectorSubcoreMesh(core_axis_name="c", subcore_axis_name="s")
    f = pl.kernel(
        functools.partial(_sc_pre_body, N=N, B=B, maxN=maxN, maxE=maxE, H=H),
        mesh=mesh,
        compiler_params=pltpu.CompilerParams(needs_layout_passes=False),
        out_type=(
            jax.ShapeDtypeStruct((M, H), jnp.float32),
            jax.ShapeDtypeStruct((B, 2 * maxE), jnp.int32),
            jax.ShapeDtypeStruct((B, 2 * maxE), jnp.int32),
            jax.ShapeDtypeStruct((M,), jnp.float32),
        ),
        scratch_types=[
            pltpu.VMEM((maxN,), jnp.int32),
            pltpu.VMEM((maxN // 128, 128), jnp.int32),
            pltpu.VMEM((N,), jnp.int32),
            pltpu.VMEM((maxE,), jnp.int32),
            pltpu.VMEM((maxE,), jnp.int32),
            pltpu.VMEM((maxN,), jnp.float32),
            pltpu.VMEM((maxN,), jnp.float32),
            pltpu.VMEM((2 * maxE,), jnp.int32),
            pltpu.VMEM((2 * maxE,), jnp.int32),
            pltpu.VMEM((128, H), jnp.float32),
            pltpu.SemaphoreType.DMA,
        ],
    )
    return f(h0, subG_node, subG_node.reshape(B, maxN // 128, 128), eu, ev)


def _sc_spmm_body(table, gdst4, gsrc4, out,
                  idxd_v, idxs_v, r0_v, r1_v, r2_v, r3_v, zbuf_v, acc_sh,
                  g0_s, g1_s, g2_s, g3_s, s0_s, s1_s, s2_s, s3_s,
                  *, B, maxN, maxE, H, grp):
    cid = lax.axis_index("c")
    sid = lax.axis_index("s")
    npass = B // (2 * grp)
    nch = grp * (2 * maxE) // 16 // 128
    rows = (r0_v, r1_v, r2_v, r3_v)
    gsem = (g0_s, g1_s, g2_s, g3_s)
    ssem = (s0_s, s1_s, s2_s, s3_s)
    zrows = zbuf_v.shape[0]
    myrows = grp * maxN // 16

    def fire_gather(c, slot):
        pltpu.async_copy(table.at[idxd_v.at[c]], rows[slot], gsem[slot])

    def wait_gather(c, slot):
        pltpu.make_async_copy(
            table.at[idxd_v.at[c]], rows[slot], gsem[slot]).wait()

    def fire_scatter(c, slot):
        pltpu.async_copy(rows[slot], acc_sh.at[idxs_v.at[c]], ssem[slot],
                         add=True)

    def wait_scatter(c, slot):
        pltpu.make_async_copy(
            rows[slot], acc_sh.at[idxs_v.at[c]], ssem[slot]).wait()

    def zb(r, _):
        for f in range(H // _L):
            zbuf_v[r, pl.ds(f * _L, _L)] = jnp.zeros((_L,), jnp.float32)
        return _
    lax.fori_loop(0, zrows, zb, None)

    for p in range(npass):
        base_sub = p * 2 * grp + cid * grp
        base_row = base_sub * maxN

        for q in range(grp):
            pltpu.sync_copy(gdst4.at[base_sub + q, sid],
                            idxd_v.at[pl.ds(q * 8, 8)])
            pltpu.sync_copy(gsrc4.at[base_sub + q, sid],
                            idxs_v.at[pl.ds(q * 8, 8)])

        def zacc(j, _):
            pltpu.sync_copy(
                zbuf_v, acc_sh.at[pl.ds(sid * myrows + j * zrows, zrows)])
            return _
        lax.fori_loop(0, myrows // zrows, zacc, None)
        plsc.subcore_barrier()

        fire_gather(0, 0)
        fire_gather(1, 1)
        fire_gather(2, 2)

        def grp_body(g, _):
            for s in range(4):
                c = g * 4 + s
                wait_gather(c, s)
                fire_scatter(c, s)

                @pl.when(c >= 1)
                def _():
                    wait_scatter(c - 1, (s + 3) % 4)

                @pl.when(c + 3 < nch)
                def _():
                    fire_gather(c + 3, (s + 3) % 4)
            return _
        lax.fori_loop(0, nch // 4, grp_body, None)

        wait_scatter(nch - 1, (nch - 1) % 4)
        plsc.subcore_barrier()

        pltpu.sync_copy(acc_sh.at[pl.ds(sid * myrows, myrows)],
                        out.at[pl.ds(base_row + sid * myrows, myrows)])
        plsc.subcore_barrier()


def _sc_spmm(table, gdst4, gsrc4, B, maxN, maxE):
    M, H = table.shape
    grp = _GRP
    nch = grp * (2 * maxE) // 16 // 128
    mesh = plsc.VectorSubcoreMesh(core_axis_name="c", subcore_axis_name="s")
    f = pl.kernel(
        functools.partial(_sc_spmm_body, B=B, maxN=maxN, maxE=maxE, H=H,
                          grp=grp),
        mesh=mesh,
        compiler_params=pltpu.CompilerParams(needs_layout_passes=False),
        out_type=jax.ShapeDtypeStruct((M, H), jnp.float32),
        scratch_types=[
            pltpu.VMEM((nch, 128), jnp.int32),
            pltpu.VMEM((nch, 128), jnp.int32),
            pltpu.VMEM((128, H), jnp.float32),
            pltpu.VMEM((128, H), jnp.float32),
            pltpu.VMEM((128, H), jnp.float32),
            pltpu.VMEM((128, H), jnp.float32),
            pltpu.VMEM((64, H), jnp.float32),
            pltpu.VMEM_SHARED((grp * maxN, H), jnp.float32),
            pltpu.SemaphoreType.DMA, pltpu.SemaphoreType.DMA,
            pltpu.SemaphoreType.DMA, pltpu.SemaphoreType.DMA,
            pltpu.SemaphoreType.DMA, pltpu.SemaphoreType.DMA,
            pltpu.SemaphoreType.DMA, pltpu.SemaphoreType.DMA,
        ],
    )
    return f(table, gdst4, gsrc4)


def kernel(x, subG_node, subG_edge, W_in, b_in, W1, W2, W_out, b_out):
    B, maxN = subG_node.shape
    maxE = subG_edge.shape[1]
    eu = subG_edge[..., 0]
    ev = subG_edge[..., 1]
    h0 = _tc_in(x, W_in, b_in)
    g0, gsrc, gdst, dinv = _sc_pre(h0, subG_node, eu, ev)
    gsrc4 = gsrc.reshape(B, 16, (2 * maxE) // 16 // 128, 128)
    gdst4 = gdst.reshape(B, 16, (2 * maxE) // 16 // 128, 128)
    out1 = _sc_spmm(g0, gdst4, gsrc4, B, maxN, maxE)
    g1 = _tc_mid(out1, dinv, W1)
    out2 = _sc_spmm(g1, gdst4, gsrc4, B, maxN, maxE)
    return _tc_out(out2, dinv, W2, W_out, b_out, B, maxN)


# speedup vs baseline: 109.3332x; 1.1228x over previous
import functools

import jax
import jax.numpy as jnp
from jax import lax
from jax.experimental import pallas as pl
from jax.experimental.pallas import tpu as pltpu
from jax.experimental.pallas import tpu_sc as plsc


def _tc_in_body(x_ref, w_ref, b_ref, o_ref):
    o_ref[...] = jax.nn.relu(
        jnp.dot(x_ref[...], w_ref[...], preferred_element_type=jnp.float32)
        + b_ref[...])


def _tc_in(x, W_in, b_in):
    N, F = x.shape
    H = W_in.shape[1]
    blk = 2000
    return pl.pallas_call(
        _tc_in_body,
        grid=(N // blk,),
        in_specs=[
            pl.BlockSpec((blk, F), lambda i: (i, 0)),
            pl.BlockSpec((F, H), lambda i: (0, 0)),
            pl.BlockSpec((1, H), lambda i: (0, 0)),
        ],
        out_specs=pl.BlockSpec((blk, H), lambda i: (i, 0)),
        out_shape=jax.ShapeDtypeStruct((N, H), jnp.float32),
    )(x, W_in, b_in.reshape(1, -1))


def _tc_mid_body(o1_ref, dinv_ref, w_ref, g_ref):
    t = o1_ref[...] * dinv_ref[...]
    h = jax.nn.relu(jnp.dot(t, w_ref[...], preferred_element_type=jnp.float32))
    g_ref[...] = h * dinv_ref[...]


def _tc_mid(out1, dinv, W1):
    M, H = out1.shape
    blk = 2048
    return pl.pallas_call(
        _tc_mid_body,
        grid=(M // blk,),
        in_specs=[
            pl.BlockSpec((blk, H), lambda i: (i, 0)),
            pl.BlockSpec((blk, 1), lambda i: (i, 0)),
            pl.BlockSpec((H, H), lambda i: (0, 0)),
        ],
        out_specs=pl.BlockSpec((blk, H), lambda i: (i, 0)),
        out_shape=jax.ShapeDtypeStruct((M, H), jnp.float32),
    )(out1, dinv.reshape(M, 1), W1)


def _tc_out_body(o2_ref, dinv_ref, w2_ref, wo_ref, bo_ref, y_ref, *, maxN):
    i = pl.program_id(0)
    t = o2_ref[...] * dinv_ref[...]
    h2 = jax.nn.relu(jnp.dot(t, w2_ref[...], preferred_element_type=jnp.float32))
    pooled = jnp.sum(h2, axis=0, keepdims=True) * (1.0 / maxN)
    y_ref[pl.ds(i, 1), :] = (
        jnp.dot(pooled, wo_ref[...], preferred_element_type=jnp.float32)
        + bo_ref[...])


def _tc_out(out2, dinv, W2, W_out, b_out, B, maxN):
    M, H = out2.shape
    OUT = W_out.shape[1]
    return pl.pallas_call(
        functools.partial(_tc_out_body, maxN=maxN),
        grid=(B,),
        in_specs=[
            pl.BlockSpec((maxN, H), lambda i: (i, 0)),
            pl.BlockSpec((maxN, 1), lambda i: (i, 0)),
            pl.BlockSpec((H, H), lambda i: (0, 0)),
            pl.BlockSpec((H, OUT), lambda i: (0, 0)),
            pl.BlockSpec((1, OUT), lambda i: (0, 0)),
        ],
        out_specs=pl.BlockSpec((B, OUT), lambda i: (0, 0)),
        out_shape=jax.ShapeDtypeStruct((B, OUT), jnp.float32),
    )(out2, dinv.reshape(M, 1), W2, W_out, b_out.reshape(1, -1))


_L = 16
_GRP = 2


def _rsqrt16(d):
    i = plsc.bitcast(d, jnp.int32)
    i = jnp.int32(0x5F3759DF) - lax.shift_right_logical(i, 1)
    y = plsc.bitcast(i, jnp.float32)
    for _ in range(3):
        y = y * (1.5 - 0.5 * d * y * y)
    return y


def _sc_pre_body(h0, nodes, nodes3, eu, ev,
                 g0, gsrc, gdst, dinv,
                 nodes_v, nodes2_v, pos_v, eu_v, ev_v,
                 glu_v, glv_v, dlu_v, dlv_v,
                 deg_v, dtmp_v, dinv_v, ra_v, rb_v, deg_sh,
                 ga_s, gb_s, wa_s, wb_s,
                 *, N, B, maxN, maxE, H):
    cid = lax.axis_index("c")
    sid = lax.axis_index("s")
    h = sid % 2
    t = sid // 2
    b = 4 * (t // 2) + 2 * cid + (t % 2)
    boff = b * maxN
    qoff = (b % _GRP) * maxN
    half = maxE // 2

    pltpu.sync_copy(nodes.at[b], nodes_v)
    pltpu.sync_copy(nodes3.at[b, pl.ds(h * 8, 8)], nodes2_v)
    pltpu.sync_copy(eu.at[b, pl.ds(h * half, half)], eu_v)
    pltpu.sync_copy(ev.at[b, pl.ds(h * half, half)], ev_v)

    def pos_body(k, _):
        idx = nodes_v[pl.ds(k * _L, _L)]
        plsc.store_scatter(pos_v, [idx], lax.iota(jnp.int32, _L) + k * _L)
        deg_v[pl.ds(k * _L, _L)] = jnp.zeros((_L,), jnp.float32)
        return _
    lax.fori_loop(0, maxN // _L, pos_body, None)

    ones = jnp.ones((_L,), jnp.float32)

    def edge_body(k, _):
        u = eu_v[pl.ds(k * _L, _L)]
        v = ev_v[pl.ds(k * _L, _L)]
        lu = plsc.load_gather(pos_v, [u])
        lv = plsc.load_gather(pos_v, [v])
        plsc.addupdate_scatter(deg_v, [lu], ones)
        plsc.addupdate_scatter(deg_v, [lv], ones)
        glu_v[pl.ds(k * _L, _L)] = lu + qoff
        glv_v[pl.ds(k * _L, _L)] = lv + qoff
        dlu_v[pl.ds(k * _L, _L)] = lu + boff
        dlv_v[pl.ds(k * _L, _L)] = lv + boff
        return _
    lax.fori_loop(0, half // _L, edge_body, None)

    pltpu.sync_copy(glu_v, gsrc.at[b, pl.ds(h * half, half)])
    pltpu.sync_copy(glv_v, gsrc.at[b, pl.ds(maxE + h * half, half)])
    pltpu.sync_copy(dlv_v, gdst.at[b, pl.ds(h * half, half)])
    pltpu.sync_copy(dlu_v, gdst.at[b, pl.ds(maxE + h * half, half)])

    pltpu.sync_copy(deg_v, deg_sh.at[t, h])
    plsc.subcore_barrier()
    pltpu.sync_copy(deg_sh.at[t, 1 - h], dtmp_v)

    def dinv_body(k, _):
        d = deg_v[pl.ds(k * _L, _L)] + dtmp_v[pl.ds(k * _L, _L)]
        d = jnp.where(d < 0.5, d + 1.0, d)
        dinv_v[pl.ds(k * _L, _L)] = _rsqrt16(d)
        return _
    lax.fori_loop(0, maxN // _L, dinv_body, None)

    @pl.when(h == 0)
    def _():
        pltpu.sync_copy(dinv_v, dinv.at[pl.ds(b * maxN, maxN)])

    rows = (ra_v, rb_v)
    gsems = (ga_s, gb_s)
    wsems = (wa_s, wb_s)
    nck = maxN // 2 // 128

    def fire_gather(j, s):
        pltpu.async_copy(h0.at[nodes2_v.at[j]], rows[s], gsems[s])

    def wait_gather(j, s):
        pltpu.make_async_copy(h0.at[nodes2_v.at[j]], rows[s],
                              gsems[s]).wait()

    def dst_of(j):
        return g0.at[pl.ds(boff + h * (maxN // 2) + j * 128, 128)]

    def fire_write(j, s):
        pltpu.async_copy(rows[s], dst_of(j), wsems[s])

    def wait_write(j, s):
        pltpu.make_async_copy(rows[s], dst_of(j), wsems[s]).wait()

    fire_gather(0, 0)

    def ck_body(jj, _):
        for s2 in range(2):
            j = jj * 2 + s2
            wait_gather(j, s2)

            @pl.when(j >= 1)
            def _():
                wait_write(j - 1, 1 - s2)

            @pl.when(j + 1 < nck)
            def _():
                fire_gather(j + 1, 1 - s2)

            rb = h * (maxN // 2) + j * 128

            def row_body(r, _):
                sc = plsc.load_gather(
                    dinv_v, [jnp.full((_L,), rb + r, jnp.int32)])
                for f in range(H // _L):
                    rows[s2][r, pl.ds(f * _L, _L)] = (
                        rows[s2][r, pl.ds(f * _L, _L)] * sc)
                return _
            lax.fori_loop(0, 128, row_body, None)
            fire_write(j, s2)
        return _
    lax.fori_loop(0, nck // 2, ck_body, None)
    wait_write(nck - 1, (nck - 1) % 2)


def _sc_pre(h0, subG_node, eu, ev):
    N, H = h0.shape
    B, maxN = subG_node.shape
    maxE = eu.shape[1]
    M = B * maxN
    mesh = plsc.VectorSubcoreMesh(core_axis_name="c", subcore_axis_name="s")
    f = pl.kernel(
        functools.partial(_sc_pre_body, N=N, B=B, maxN=maxN, maxE=maxE, H=H),
        mesh=mesh,
        compiler_params=pltpu.CompilerParams(needs_layout_passes=False),
        out_type=(
            jax.ShapeDtypeStruct((M, H), jnp.float32),
            jax.ShapeDtypeStruct((B, 2 * maxE), jnp.int32),
            jax.ShapeDtypeStruct((B, 2 * maxE), jnp.int32),
            jax.ShapeDtypeStruct((M,), jnp.float32),
        ),
        scratch_types=[
            pltpu.VMEM((maxN,), jnp.int32),
            pltpu.VMEM((8, 128), jnp.int32),
            pltpu.VMEM((N,), jnp.int32),
            pltpu.VMEM((maxE // 2,), jnp.int32),
            pltpu.VMEM((maxE // 2,), jnp.int32),
            pltpu.VMEM((maxE // 2,), jnp.int32),
            pltpu.VMEM((maxE // 2,), jnp.int32),
            pltpu.VMEM((maxE // 2,), jnp.int32),
            pltpu.VMEM((maxE // 2,), jnp.int32),
            pltpu.VMEM((maxN,), jnp.float32),
            pltpu.VMEM((maxN,), jnp.float32),
            pltpu.VMEM((maxN,), jnp.float32),
            pltpu.VMEM((128, H), jnp.float32),
            pltpu.VMEM((128, H), jnp.float32),
            pltpu.VMEM_SHARED((8, 2, maxN), jnp.float32),
            pltpu.SemaphoreType.DMA, pltpu.SemaphoreType.DMA,
            pltpu.SemaphoreType.DMA, pltpu.SemaphoreType.DMA,
        ],
    )
    return f(h0, subG_node, subG_node.reshape(B, maxN // 128, 128), eu, ev)


def _sc_spmm_body(table, gdst4, gsrc4, out,
                  idxd_v, idxs_v, r0_v, r1_v, r2_v, r3_v, zbuf_v, acc_sh,
                  g0_s, g1_s, g2_s, g3_s, s0_s, s1_s, s2_s, s3_s,
                  *, B, maxN, maxE, H, grp):
    cid = lax.axis_index("c")
    sid = lax.axis_index("s")
    npass = B // (2 * grp)
    nch = grp * (2 * maxE) // 16 // 128
    rows = (r0_v, r1_v, r2_v, r3_v)
    gsem = (g0_s, g1_s, g2_s, g3_s)
    ssem = (s0_s, s1_s, s2_s, s3_s)
    zrows = zbuf_v.shape[0]
    myrows = grp * maxN // 16

    def fire_gather(c, slot):
        pltpu.async_copy(table.at[idxd_v.at[c]], rows[slot], gsem[slot])

    def wait_gather(c, slot):
        pltpu.make_async_copy(
            table.at[idxd_v.at[c]], rows[slot], gsem[slot]).wait()

    def fire_scatter(c, slot):
        pltpu.async_copy(rows[slot], acc_sh.at[idxs_v.at[c]], ssem[slot],
                         add=True)

    def wait_scatter(c, slot):
        pltpu.make_async_copy(
            rows[slot], acc_sh.at[idxs_v.at[c]], ssem[slot]).wait()

    def zb(r, _):
        for f in range(H // _L):
            zbuf_v[r, pl.ds(f * _L, _L)] = jnp.zeros((_L,), jnp.float32)
        return _
    lax.fori_loop(0, zrows, zb, None)

    for p in range(npass):
        base_sub = p * 2 * grp + cid * grp
        base_row = base_sub * maxN

        for q in range(grp):
            pltpu.sync_copy(gdst4.at[base_sub + q, sid],
                            idxd_v.at[pl.ds(q * 8, 8)])
            pltpu.sync_copy(gsrc4.at[base_sub + q, sid],
                            idxs_v.at[pl.ds(q * 8, 8)])

        def zacc(j, _):
            pltpu.sync_copy(
                zbuf_v, acc_sh.at[pl.ds(sid * myrows + j * zrows, zrows)])
            return _
        lax.fori_loop(0, myrows // zrows, zacc, None)
        plsc.subcore_barrier()

        fire_gather(0, 0)
        fire_gather(1, 1)
        fire_gather(2, 2)

        def grp_body(g, _):
            for s in range(4):
                c = g * 4 + s
                wait_gather(c, s)
                fire_scatter(c, s)

                @pl.when(c >= 1)
                def _():
                    wait_scatter(c - 1, (s + 3) % 4)

                @pl.when(c + 3 < nch)
                def _():
                    fire_gather(c + 3, (s + 3) % 4)
            return _
        lax.fori_loop(0, nch // 4, grp_body, None)

        wait_scatter(nch - 1, (nch - 1) % 4)
        plsc.subcore_barrier()

        pltpu.sync_copy(acc_sh.at[pl.ds(sid * myrows, myrows)],
                        out.at[pl.ds(base_row + sid * myrows, myrows)])
        plsc.subcore_barrier()


def _sc_spmm(table, gdst4, gsrc4, B, maxN, maxE):
    M, H = table.shape
    grp = _GRP
    nch = grp * (2 * maxE) // 16 // 128
    mesh = plsc.VectorSubcoreMesh(core_axis_name="c", subcore_axis_name="s")
    f = pl.kernel(
        functools.partial(_sc_spmm_body, B=B, maxN=maxN, maxE=maxE, H=H,
                          grp=grp),
        mesh=mesh,
        compiler_params=pltpu.CompilerParams(needs_layout_passes=False),
        out_type=jax.ShapeDtypeStruct((M, H), jnp.float32),
        scratch_types=[
            pltpu.VMEM((nch, 128), jnp.int32),
            pltpu.VMEM((nch, 128), jnp.int32),
            pltpu.VMEM((128, H), jnp.float32),
            pltpu.VMEM((128, H), jnp.float32),
            pltpu.VMEM((128, H), jnp.float32),
            pltpu.VMEM((128, H), jnp.float32),
            pltpu.VMEM((64, H), jnp.float32),
            pltpu.VMEM_SHARED((grp * maxN, H), jnp.float32),
            pltpu.SemaphoreType.DMA, pltpu.SemaphoreType.DMA,
            pltpu.SemaphoreType.DMA, pltpu.SemaphoreType.DMA,
            pltpu.SemaphoreType.DMA, pltpu.SemaphoreType.DMA,
            pltpu.SemaphoreType.DMA, pltpu.SemaphoreType.DMA,
        ],
    )
    return f(table, gdst4, gsrc4)


def kernel(x, subG_node, subG_edge, W_in, b_in, W1, W2, W_out, b_out):
    B, maxN = subG_node.shape
    maxE = subG_edge.shape[1]
    eu = subG_edge[..., 0]
    ev = subG_edge[..., 1]
    h0 = _tc_in(x, W_in, b_in)
    g0, gsrc, gdst, dinv = _sc_pre(h0, subG_node, eu, ev)
    gsrc4 = gsrc.reshape(B, 16, (2 * maxE) // 16 // 128, 128)
    gdst4 = gdst.reshape(B, 16, (2 * maxE) // 16 // 128, 128)
    out1 = _sc_spmm(g0, gdst4, gsrc4, B, maxN, maxE)
    g1 = _tc_mid(out1, dinv, W1)
    out2 = _sc_spmm(g1, gdst4, gsrc4, B, maxN, maxE)
    return _tc_out(out2, dinv, W2, W_out, b_out, B, maxN)
